# baseline (speedup 1.0000x reference)
import jax
import jax.numpy as jnp
from jax import lax
from jax.experimental import pallas as pl
from jax.experimental.pallas import tpu as pltpu
from jax.experimental.pallas import tpu_sc as plsc

DIM = 32
BATCH = 16384
HIST = 20
N = BATCH * HIST

NC = 2
NS = 16
NW = NC * NS

PER_W = N // NW
B_PER_W = BATCH // NW
SUB = 128
ROWS_PER_W = PER_W // SUB
B_CHUNK = 32
CHUNK = B_CHUNK * HIST
SUB_PER_CHUNK = CHUNK // SUB
NCHUNK = PER_W // CHUNK
NPAIR = NCHUNK // 2
L = 16


def _body(idx1_hbm, idx2_hbm, t1_hbm, t2_hbm, out_hbm,
          idx1_v, idx2_v, rows_a, rows_b, trans_a, trans_b,
          sem_a, sem_b, sem_o):
    c = lax.axis_index("c")
    s = lax.axis_index("s")
    wid = s * NC + c
    pltpu.sync_copy(idx1_hbm.at[wid], idx1_v)
    pltpu.sync_copy(idx2_hbm.at[wid], idx2_v)
    b_base = wid * B_PER_W

    iota = lax.iota(jnp.int32, L)

    def t1_descs(rows_v, sem, ci):
        out = []
        for j in range(SUB_PER_CHUNK):
            row = ci * SUB_PER_CHUNK + j
            dst = pl.ds(j * SUB, SUB)
            out.append((t1_hbm.at[idx1_v.at[row]], rows_v.at[dst], sem))
        return out

    def fire(descs):
        for src, dst, sem in descs:
            pltpu.async_copy(src, dst, sem)

    def drain(descs):
        for src, dst, sem in descs:
            pltpu.make_async_copy(src, dst, sem).wait()

    def add_gathers(rows_v, sem, ci):
        descs = []
        for j in range(SUB_PER_CHUNK):
            row = ci * SUB_PER_CHUNK + j
            dst = pl.ds(j * SUB, SUB)
            descs.append(pltpu.async_copy(
                t2_hbm.at[idx2_v.at[row]], rows_v.at[dst], sem, add=True))
        for d in descs:
            d.wait()

    def out_slice(ci):
        return out_hbm.at[:, :, pl.ds(b_base + ci * B_CHUNK, B_CHUNK)]

    def transpose(rows_v, trans_v):
        def tr_body(b_loc, _):
            for h in range(HIST):
                l = b_loc * HIST + h
                h_vec = lax.full((L,), h, jnp.int32)
                b_vec = lax.full((L,), 0, jnp.int32) + b_loc
                for dblock in range(DIM // L):
                    x = rows_v[l, pl.ds(dblock * L, L)]
                    d_vec = dblock * L + iota
                    plsc.store_scatter(trans_v, [h_vec, d_vec, b_vec], x)
            return ()

        lax.fori_loop(0, B_CHUNK, tr_body, ())

    def trans_src(trans_v):
        return trans_v.at[:, :, pl.ds(0, B_CHUNK)]

    fire(t1_descs(rows_a, sem_a, 0))

    def pair_body(p, _):
        e = 2 * p
        o = e + 1
        drain(t1_descs(rows_a, sem_a, e))
        add_gathers(rows_a, sem_a, e)
        fire(t1_descs(rows_b, sem_b, o))
        @pl.when(p > 0)
        def _():
            pltpu.make_async_copy(
                trans_src(trans_a), out_slice(e - 2), sem_o).wait()
        transpose(rows_a, trans_a)
        pltpu.async_copy(trans_src(trans_a), out_slice(e), sem_o)
        drain(t1_descs(rows_b, sem_b, o))
        add_gathers(rows_b, sem_b, o)
        @pl.when(p + 1 < NPAIR)
        def _():
            fire(t1_descs(rows_a, sem_a, (e + 2) % NCHUNK))
        @pl.when(p > 0)
        def _():
            pltpu.make_async_copy(
                trans_src(trans_b), out_slice(o - 2), sem_o).wait()
        transpose(rows_b, trans_b)
        pltpu.async_copy(trans_src(trans_b), out_slice(o), sem_o)
        return ()

    lax.fori_loop(0, NPAIR, pair_body, ())
    pltpu.make_async_copy(
        trans_src(trans_a), out_slice(NCHUNK - 2), sem_o).wait()
    pltpu.make_async_copy(
        trans_src(trans_b), out_slice(NCHUNK - 1), sem_o).wait()


def kernel(input, another_input, table1, table2):
    idx1 = input.reshape(-1).astype(jnp.int32).reshape(NW, ROWS_PER_W, SUB)
    idx2 = another_input.reshape(-1).astype(jnp.int32).reshape(NW, ROWS_PER_W, SUB)
    mesh = plsc.VectorSubcoreMesh(core_axis_name="c", subcore_axis_name="s")
    out5 = pl.kernel(
        _body,
        out_type=jax.ShapeDtypeStruct((HIST, DIM, BATCH), jnp.float32),
        mesh=mesh,
        compiler_params=pltpu.CompilerParams(
            use_tc_tiling_on_sc=False, needs_layout_passes=False),
        scratch_types=[
            pltpu.VMEM((ROWS_PER_W, SUB), jnp.int32),
            pltpu.VMEM((ROWS_PER_W, SUB), jnp.int32),
            pltpu.VMEM((CHUNK, DIM), jnp.float32),
            pltpu.VMEM((CHUNK, DIM), jnp.float32),
            pltpu.VMEM((HIST, DIM, B_CHUNK + 1), jnp.float32),
            pltpu.VMEM((HIST, DIM, B_CHUNK + 1), jnp.float32),
            pltpu.SemaphoreType.DMA,
            pltpu.SemaphoreType.DMA,
            pltpu.SemaphoreType.DMA,
        ],
    )(idx1, idx2, table1, table2)
    return out5.transpose(2, 0, 1)


# speedup vs baseline: 1.0062x; 1.0062x over previous
import jax
import jax.numpy as jnp
from jax import lax
from jax.experimental import pallas as pl
from jax.experimental.pallas import tpu as pltpu
from jax.experimental.pallas import tpu_sc as plsc

DIM = 32
BATCH = 16384
HIST = 20
N = BATCH * HIST

NC = 2
NS = 16
NW = NC * NS

PER_W = N // NW
B_PER_W = BATCH // NW
SUB = 128
ROWS_PER_W = PER_W // SUB
B_CHUNK = 64
CHUNK = B_CHUNK * HIST
SUB_PER_CHUNK = CHUNK // SUB
NCHUNK = PER_W // CHUNK
L = 16


def _body(idx1_hbm, idx2_hbm, t1_hbm, t2_hbm, out_hbm,
          idx1_v, idx2_v, rows_v, trans_v, sem):
    c = lax.axis_index("c")
    s = lax.axis_index("s")
    wid = s * NC + c
    pltpu.sync_copy(idx1_hbm.at[wid], idx1_v)
    pltpu.sync_copy(idx2_hbm.at[wid], idx2_v)
    b_base = wid * B_PER_W

    iota = lax.iota(jnp.int32, L)

    def chunk_body(ci, _):
        descs = []
        for j in range(SUB_PER_CHUNK):
            row = ci * SUB_PER_CHUNK + j
            dst = pl.ds(j * SUB, SUB)
            descs.append(pltpu.async_copy(
                t1_hbm.at[idx1_v.at[row]], rows_v.at[dst], sem))
        for d in descs:
            d.wait()
        descs = []
        for j in range(SUB_PER_CHUNK):
            row = ci * SUB_PER_CHUNK + j
            dst = pl.ds(j * SUB, SUB)
            descs.append(pltpu.async_copy(
                t2_hbm.at[idx2_v.at[row]], rows_v.at[dst], sem, add=True))
        for d in descs:
            d.wait()

        def tr_body(b_loc, _):
            for h in range(HIST):
                l = b_loc * HIST + h
                h_vec = lax.full((L,), h, jnp.int32)
                b_vec = lax.full((L,), 0, jnp.int32) + b_loc
                for dblock in range(DIM // L):
                    x = rows_v[l, pl.ds(dblock * L, L)]
                    d_vec = dblock * L + iota
                    plsc.store_scatter(trans_v, [h_vec, d_vec, b_vec], x)
            return ()

        lax.fori_loop(0, B_CHUNK, tr_body, ())
        pltpu.sync_copy(
            trans_v.at[:, :, pl.ds(0, B_CHUNK)],
            out_hbm.at[:, :, pl.ds(b_base + ci * B_CHUNK, B_CHUNK)])
        return ()

    lax.fori_loop(0, NCHUNK, chunk_body, ())


def kernel(input, another_input, table1, table2):
    idx1 = input.reshape(-1).astype(jnp.int32).reshape(NW, ROWS_PER_W, SUB)
    idx2 = another_input.reshape(-1).astype(jnp.int32).reshape(NW, ROWS_PER_W, SUB)
    mesh = plsc.VectorSubcoreMesh(core_axis_name="c", subcore_axis_name="s")
    out5 = pl.kernel(
        _body,
        out_type=jax.ShapeDtypeStruct((HIST, DIM, BATCH), jnp.float32),
        mesh=mesh,
        compiler_params=pltpu.CompilerParams(
            use_tc_tiling_on_sc=False, needs_layout_passes=False),
        scratch_types=[
            pltpu.VMEM((ROWS_PER_W, SUB), jnp.int32),
            pltpu.VMEM((ROWS_PER_W, SUB), jnp.int32),
            pltpu.VMEM((CHUNK, DIM), jnp.float32),
            pltpu.VMEM((HIST, DIM, B_CHUNK + 1), jnp.float32),
            pltpu.SemaphoreType.DMA,
        ],
    )(idx1, idx2, table1, table2)
    return out5.transpose(2, 0, 1)
